# baseline (speedup 1.0000x reference)
import functools

import jax
import jax.numpy as jnp
from jax import lax
from jax.experimental import pallas as pl
from jax.experimental.pallas import tpu as pltpu
from jax.experimental.pallas import tpu_sc as plsc

_NV = 100000
_NF = 200000
_CIN = 128
_COUT = 128
_K = 8
_BF = 8000
_BV = 5000

_VPP = 12544
_ACC_ROWS = 12552
_DUMMY = 12544
_PASSES = 4
_COV = _PASSES * 2 * _VPP
_FPT = 12544
_CCH = 1568
_NSLOT = 128
_FPAD = _NSLOT * _CCH


def _facet_body(filt_ref, x_ref, w_ref, tmp_ref):
    w = jnp.dot(filt_ref[...], w_ref[...], preferred_element_type=jnp.float32)
    tmp_ref[...] = w * x_ref[...]


def _vert_body(acc_ref, cnt_ref, wd_ref, b_ref, pre_ref, stats_ref):
    i = pl.program_id(0)
    denom = jnp.maximum(cnt_ref[0, 0, :], 1).astype(jnp.float32)
    vert = acc_ref[...] / denom[:, None]
    pre = jnp.dot(vert, wd_ref[...], preferred_element_type=jnp.float32)
    pre = jnp.maximum(pre + b_ref[...], 0.0)
    pre_ref[...] = pre

    @pl.when(i == 0)
    def _():
        stats_ref[...] = jnp.zeros_like(stats_ref)

    s1 = jnp.sum(pre, axis=0, keepdims=True)
    s2 = jnp.sum(pre * pre, axis=0, keepdims=True)
    pad = jnp.zeros((6, _COUT), dtype=jnp.float32)
    stats_ref[...] += jnp.concatenate([s1, s2, pad], axis=0)


def _norm_body(pre_ref, stats_ref, g_ref, b_ref, out_ref):
    mean = stats_ref[0:1, :] / _NV
    ex2 = stats_ref[1:2, :] / _NV
    var = ex2 - mean * mean
    rstd = jax.lax.rsqrt(var + 1e-5)
    out_ref[...] = (pre_ref[...] - mean) * rstd * g_ref[...] + b_ref[...]


def _facet_weight(inputs, filt_coeff, sw2d):
    grid = (_NF // _BF,)
    return pl.pallas_call(
        _facet_body,
        grid=grid,
        in_specs=[
            pl.BlockSpec((_BF, _K), lambda i: (i, 0)),
            pl.BlockSpec((_BF, _CIN), lambda i: (i, 0)),
            pl.BlockSpec((_K, _CIN), lambda i: (0, 0)),
        ],
        out_specs=pl.BlockSpec((_BF, _CIN), lambda i: (i, 0)),
        out_shape=jax.ShapeDtypeStruct((_NF, _CIN), jnp.float32),
    )(filt_coeff, inputs, sw2d)


def _sc_scatter(tmp, face_t):
    mesh = plsc.VectorSubcoreMesh(core_axis_name="c", subcore_axis_name="s")

    @functools.partial(
        pl.kernel,
        out_type=jax.ShapeDtypeStruct((_COV, _CIN), jnp.float32),
        mesh=mesh,
        compiler_params=pltpu.CompilerParams(needs_layout_passes=False),
        scratch_types=[
            pltpu.VMEM((2 * 3 * _CCH,), jnp.int32),
            pltpu.VMEM((192,), jnp.int32),
            pltpu.VMEM((192,), jnp.int32),
            pltpu.VMEM((64,), jnp.int32),
            pltpu.VMEM((64,), jnp.int32),
            pltpu.VMEM((64,), jnp.int32),
            pltpu.VMEM((64,), jnp.int32),
            pltpu.VMEM((128, _CIN), jnp.float32),
            pltpu.VMEM_SHARED((_ACC_ROWS, _CIN), jnp.float32),
            pltpu.SemaphoreType.DMA,
            pltpu.SemaphoreType.DMA,
            pltpu.SemaphoreType.DMA,
        ],
    )
    def k(tmp_hbm, face_hbm, out_hbm, colbuf, sfid, slv,
          gidx0, sidx0, gidx1, sidx1, rowbuf, acc, sem0, sem1, csem):
        cid = lax.axis_index("c")
        sid = lax.axis_index("s")
        iota = lax.iota(jnp.int32, 16)
        zero16f = jnp.zeros((16,), jnp.float32)

        fstart = sid * _FPT
        nmy = jnp.minimum(_FPT, _NF - fstart)
        nchunks = (nmy + _CCH - 1) // _CCH

        halves = ((gidx0, sidx0, 0, sem0), (gidx1, sidx1, 64, sem1))

        def drain_half(q):
            gx, sx, rb, sm = halves[q]
            pltpu.make_async_copy(tmp_hbm.at[gx],
                                  rowbuf.at[pl.ds(rb, 64)], sm).wait()
            pltpu.sync_copy(rowbuf.at[pl.ds(rb, 64)], acc.at[sx],
                            add=True)

        def fire(fcnt):
            for q in range(2):
                @pl.when((fcnt & 1) == q)
                def _(q=q):
                    gx, sx, rb, sm = halves[q]

                    @pl.when(fcnt >= 2)
                    def _():
                        drain_half(q)
                    for off in range(0, 64, 16):
                        gx[pl.ds(off, 16)] = sfid[pl.ds(off, 16)]
                        sx[pl.ds(off, 16)] = slv[pl.ds(off, 16)]
                    pltpu.async_copy(tmp_hbm.at[gx],
                                     rowbuf.at[pl.ds(rb, 64)], sm)

        for p in range(_PASSES):
            gbase = (p * 2 + cid) * _VPP

            def zb(i, carry):
                for j in range(8):
                    rowbuf[i, pl.ds(j * 16, 16)] = zero16f
                return carry
            lax.fori_loop(0, 128, zb, 0)

            def z(j, carry):
                i = sid + j * 16

                @pl.when(i < _VPP // 128)
                def _():
                    pltpu.sync_copy(rowbuf, acc.at[pl.ds(i * 128, 128)])
                return carry
            lax.fori_loop(0, 7, z, 0)

            @pl.when(sid == 0)
            def _():
                pltpu.sync_copy(rowbuf.at[pl.ds(0, 8)],
                                acc.at[pl.ds(_VPP, 8)])
            plsc.subcore_barrier()

            def cprefetch(c, half):
                slot = sid * 8 + c
                pltpu.async_copy(
                    face_hbm.at[pl.ds(slot * 3 * _CCH, 3 * _CCH)],
                    colbuf.at[pl.ds(half * (3 * _CCH), 3 * _CCH)], csem)

            cprefetch(jnp.int32(0), jnp.int32(0))

            def chunk_body(c, carry):
                half = c % 2
                base = half * (3 * _CCH)
                pltpu.make_async_copy(
                    face_hbm.at[pl.ds(0, 3 * _CCH)],
                    colbuf.at[pl.ds(0, 3 * _CCH)], csem).wait()

                @pl.when(c + 1 < nchunks)
                def _():
                    cprefetch(c + 1, 1 - half)
                cs = fstart + c * _CCH
                ng = jnp.minimum(_CCH, nmy - c * _CCH) // 16

                def group_body(g, carry2):
                    ptrv, fcnt = carry2
                    fidv = cs + g * 16 + iota
                    for j in range(3):
                        v = colbuf[pl.ds(base + j * _CCH + g * 16, 16)]
                        lv = v - gbase
                        mask = (lv >= 0) & (lv < _VPP)
                        idxv = jnp.where(mask, lv, _DUMMY)
                        mcount = plsc.cumsum(mask.astype(jnp.int32))
                        cnt = plsc.all_reduce_population_count(mask)
                        pos = ptrv + mcount - 1
                        plsc.store_scatter(sfid, [pos], fidv, mask=mask)
                        plsc.store_scatter(slv, [pos], idxv, mask=mask)
                        ptrv = ptrv + cnt
                    do = ptrv[0] >= 64

                    @pl.when(do)
                    def _():
                        fire(fcnt)
                        for off in range(0, 48, 16):
                            a = sfid[pl.ds(64 + off, 16)]
                            b = slv[pl.ds(64 + off, 16)]
                            sfid[pl.ds(off, 16)] = a
                            slv[pl.ds(off, 16)] = b
                    dov = ptrv >= 64
                    ptrv = jnp.where(dov, ptrv - 64, ptrv)
                    fcnt = jnp.where(do, fcnt + 1, fcnt)
                    return ptrv, fcnt
                return lax.fori_loop(0, ng, group_body, carry)

            zv = jnp.zeros((16,), jnp.int32)
            ptrv, fcnt = lax.fori_loop(0, nchunks, chunk_body,
                                       (zv, jnp.int32(0)))
            ptr = ptrv[0]

            @pl.when(ptr > 0)
            def _():
                for off in range(0, 64, 16):
                    m = (off + iota) < ptr
                    fv = jnp.where(m, sfid[pl.ds(off, 16)], 0)
                    lvv = jnp.where(m, slv[pl.ds(off, 16)], _DUMMY)
                    sfid[pl.ds(off, 16)] = fv
                    slv[pl.ds(off, 16)] = lvv
                fire(fcnt)
            fcnt = fcnt + (ptr > 0).astype(jnp.int32)

            for q in range(2):
                @pl.when((fcnt >= 2) & ((fcnt & 1) == q))
                def _(q=q):
                    drain_half(q)
            for q in range(2):
                @pl.when((fcnt >= 1) & (((fcnt - 1) & 1) == q))
                def _(q=q):
                    drain_half(q)
            plsc.subcore_barrier()

            def w(j, carry):
                i = sid + j * 16

                @pl.when(i < _VPP // 128)
                def _():
                    pltpu.sync_copy(acc.at[pl.ds(i * 128, 128)],
                                    out_hbm.at[pl.ds(gbase + i * 128, 128)])
                return carry
            lax.fori_loop(0, 7, w, 0)
            plsc.subcore_barrier()

    return k(tmp, face_t)


def _vertex_stage(acc, cnt3, depth_weights, biases):
    grid = (_NV // _BV,)
    return pl.pallas_call(
        _vert_body,
        grid=grid,
        in_specs=[
            pl.BlockSpec((_BV, _CIN), lambda i: (i, 0)),
            pl.BlockSpec((1, 1, _BV), lambda i: (i, 0, 0)),
            pl.BlockSpec((_CIN, _COUT), lambda i: (0, 0)),
            pl.BlockSpec((1, _COUT), lambda i: (0, 0)),
        ],
        out_specs=[
            pl.BlockSpec((_BV, _COUT), lambda i: (i, 0)),
            pl.BlockSpec((8, _COUT), lambda i: (0, 0)),
        ],
        out_shape=[
            jax.ShapeDtypeStruct((_NV, _COUT), jnp.float32),
            jax.ShapeDtypeStruct((8, _COUT), jnp.float32),
        ],
    )(acc, cnt3, depth_weights, biases)


def _normalize(pre, stats, gamma, beta):
    grid = (_NV // _BV,)
    return pl.pallas_call(
        _norm_body,
        grid=grid,
        in_specs=[
            pl.BlockSpec((_BV, _COUT), lambda i: (i, 0)),
            pl.BlockSpec((8, _COUT), lambda i: (0, 0)),
            pl.BlockSpec((1, _COUT), lambda i: (0, 0)),
            pl.BlockSpec((1, _COUT), lambda i: (0, 0)),
        ],
        out_specs=pl.BlockSpec((_BV, _COUT), lambda i: (i, 0)),
        out_shape=jax.ShapeDtypeStruct((_NV, _COUT), jnp.float32),
    )(pre, stats, gamma, beta)


def kernel(inputs, face, nf_count, vt_map, filt_coeff, spatial_weights,
           depth_weights, biases, gamma, beta):
    del vt_map
    sw2d = spatial_weights.reshape(_K, _CIN)
    tmp = _facet_weight(inputs, filt_coeff, sw2d)

    face_t = jnp.pad(face.T, ((0, 0), (0, _FPAD - _NF)))
    face_c = face_t.reshape(3, _NSLOT, _CCH).transpose(1, 0, 2).reshape(-1)
    acc = _sc_scatter(tmp, face_c)

    cnt3 = nf_count.reshape(_NV // _BV, 1, _BV)
    pre, stats = _vertex_stage(acc, cnt3, depth_weights, biases)
    out = _normalize(pre, stats, gamma.reshape(1, _COUT), beta.reshape(1, _COUT))
    return out


# speedup vs baseline: 1.0883x; 1.0883x over previous
import functools

import jax
import jax.numpy as jnp
from jax import lax
from jax.experimental import pallas as pl
from jax.experimental.pallas import tpu as pltpu
from jax.experimental.pallas import tpu_sc as plsc

_NV = 100000
_NF = 200000
_CIN = 128
_COUT = 128
_K = 8
_BF = 8000
_BV = 5000

_VPP = 12544
_ACC_ROWS = 12552
_DUMMY = 12544
_PASSES = 4
_COV = _PASSES * 2 * _VPP
_FPT = 12544
_CCH = 1568
_NSLOT = 128
_FPAD = _NSLOT * _CCH


def _facet_body(filt_ref, x_ref, w_ref, tmp_ref):
    w = jnp.dot(filt_ref[...], w_ref[...], preferred_element_type=jnp.float32)
    tmp_ref[...] = w * x_ref[...]


def _vert_body(acc_ref, cnt_ref, wd_ref, b_ref, pre_ref, stats_ref):
    i = pl.program_id(0)
    denom = jnp.maximum(cnt_ref[0, 0, :], 1).astype(jnp.float32)
    vert = acc_ref[...] / denom[:, None]
    pre = jnp.dot(vert, wd_ref[...], preferred_element_type=jnp.float32)
    pre = jnp.maximum(pre + b_ref[...], 0.0)
    pre_ref[...] = pre

    @pl.when(i == 0)
    def _():
        stats_ref[...] = jnp.zeros_like(stats_ref)

    s1 = jnp.sum(pre, axis=0, keepdims=True)
    s2 = jnp.sum(pre * pre, axis=0, keepdims=True)
    pad = jnp.zeros((6, _COUT), dtype=jnp.float32)
    stats_ref[...] += jnp.concatenate([s1, s2, pad], axis=0)


def _norm_body(pre_ref, stats_ref, g_ref, b_ref, out_ref):
    mean = stats_ref[0:1, :] / _NV
    ex2 = stats_ref[1:2, :] / _NV
    var = ex2 - mean * mean
    rstd = jax.lax.rsqrt(var + 1e-5)
    out_ref[...] = (pre_ref[...] - mean) * rstd * g_ref[...] + b_ref[...]


def _facet_weight(inputs, filt_coeff, sw2d):
    grid = (_NF // _BF,)
    return pl.pallas_call(
        _facet_body,
        grid=grid,
        in_specs=[
            pl.BlockSpec((_BF, _K), lambda i: (i, 0)),
            pl.BlockSpec((_BF, _CIN), lambda i: (i, 0)),
            pl.BlockSpec((_K, _CIN), lambda i: (0, 0)),
        ],
        out_specs=pl.BlockSpec((_BF, _CIN), lambda i: (i, 0)),
        out_shape=jax.ShapeDtypeStruct((_NF, _CIN), jnp.float32),
    )(filt_coeff, inputs, sw2d)


def _sc_scatter(tmp, face_t):
    mesh = plsc.VectorSubcoreMesh(core_axis_name="c", subcore_axis_name="s")

    @functools.partial(
        pl.kernel,
        out_type=jax.ShapeDtypeStruct((_COV, _CIN), jnp.float32),
        mesh=mesh,
        compiler_params=pltpu.CompilerParams(needs_layout_passes=False),
        scratch_types=[
            pltpu.VMEM((2 * 3 * _CCH,), jnp.int32),
            pltpu.VMEM((192,), jnp.int32),
            pltpu.VMEM((192,), jnp.int32),
            pltpu.VMEM((32,), jnp.int32),
            pltpu.VMEM((32,), jnp.int32),
            pltpu.VMEM((32,), jnp.int32),
            pltpu.VMEM((32,), jnp.int32),
            pltpu.VMEM((32,), jnp.int32),
            pltpu.VMEM((32,), jnp.int32),
            pltpu.VMEM((32,), jnp.int32),
            pltpu.VMEM((32,), jnp.int32),
            pltpu.VMEM((128, _CIN), jnp.float32),
            pltpu.VMEM_SHARED((_ACC_ROWS, _CIN), jnp.float32),
            pltpu.SemaphoreType.DMA,
            pltpu.SemaphoreType.DMA,
            pltpu.SemaphoreType.DMA,
            pltpu.SemaphoreType.DMA,
            pltpu.SemaphoreType.DMA,
        ],
    )
    def k(tmp_hbm, face_hbm, out_hbm, colbuf, sfid, slv,
          gidx0, sidx0, gidx1, sidx1, gidx2, sidx2, gidx3, sidx3,
          rowbuf, acc, sem0, sem1, sem2, sem3, csem):
        cid = lax.axis_index("c")
        sid = lax.axis_index("s")
        iota = lax.iota(jnp.int32, 16)
        zero16f = jnp.zeros((16,), jnp.float32)

        fstart = sid * _FPT
        nmy = jnp.minimum(_FPT, _NF - fstart)
        nchunks = (nmy + _CCH - 1) // _CCH

        halves = ((gidx0, sidx0, 0, sem0), (gidx1, sidx1, 32, sem1),
                  (gidx2, sidx2, 64, sem2), (gidx3, sidx3, 96, sem3))

        def drain_half(q):
            gx, sx, rb, sm = halves[q]
            pltpu.make_async_copy(tmp_hbm.at[gx],
                                  rowbuf.at[pl.ds(rb, 32)], sm).wait()
            pltpu.sync_copy(rowbuf.at[pl.ds(rb, 32)], acc.at[sx],
                            add=True)

        def fire(fcnt):
            for q in range(4):
                @pl.when((fcnt & 3) == q)
                def _(q=q):
                    gx, sx, rb, sm = halves[q]

                    @pl.when(fcnt >= 4)
                    def _():
                        drain_half(q)
                    for off in range(0, 32, 16):
                        gx[pl.ds(off, 16)] = sfid[pl.ds(off, 16)]
                        sx[pl.ds(off, 16)] = slv[pl.ds(off, 16)]
                    pltpu.async_copy(tmp_hbm.at[gx],
                                     rowbuf.at[pl.ds(rb, 32)], sm)

        for p in range(_PASSES):
            gbase = (p * 2 + cid) * _VPP

            def zb(i, carry):
                for j in range(8):
                    rowbuf[i, pl.ds(j * 16, 16)] = zero16f
                return carry
            lax.fori_loop(0, 128, zb, 0)

            def z(j, carry):
                i = sid + j * 16

                @pl.when(i < _VPP // 128)
                def _():
                    pltpu.sync_copy(rowbuf, acc.at[pl.ds(i * 128, 128)])
                return carry
            lax.fori_loop(0, 7, z, 0)

            @pl.when(sid == 0)
            def _():
                pltpu.sync_copy(rowbuf.at[pl.ds(0, 8)],
                                acc.at[pl.ds(_VPP, 8)])
            plsc.subcore_barrier()

            def cprefetch(c, half):
                slot = sid * 8 + c
                pltpu.async_copy(
                    face_hbm.at[pl.ds(slot * 3 * _CCH, 3 * _CCH)],
                    colbuf.at[pl.ds(half * (3 * _CCH), 3 * _CCH)], csem)

            cprefetch(jnp.int32(0), jnp.int32(0))

            def chunk_body(c, carry):
                half = c % 2
                base = half * (3 * _CCH)
                pltpu.make_async_copy(
                    face_hbm.at[pl.ds(0, 3 * _CCH)],
                    colbuf.at[pl.ds(0, 3 * _CCH)], csem).wait()

                @pl.when(c + 1 < nchunks)
                def _():
                    cprefetch(c + 1, 1 - half)
                cs = fstart + c * _CCH
                ng = jnp.minimum(_CCH, nmy - c * _CCH) // 16

                def group_body(g, carry2):
                    ptrv, fcnt = carry2
                    fidv = cs + g * 16 + iota
                    for j in range(3):
                        v = colbuf[pl.ds(base + j * _CCH + g * 16, 16)]
                        lv = v - gbase
                        mask = (lv >= 0) & (lv < _VPP)
                        idxv = jnp.where(mask, lv, _DUMMY)
                        mcount = plsc.cumsum(mask.astype(jnp.int32))
                        cnt = plsc.all_reduce_population_count(mask)
                        pos = ptrv + mcount - 1
                        plsc.store_scatter(sfid, [pos], fidv, mask=mask)
                        plsc.store_scatter(slv, [pos], idxv, mask=mask)
                        ptrv = ptrv + cnt
                    do = ptrv[0] >= 32

                    @pl.when(do)
                    def _():
                        fire(fcnt)
                        for off in range(0, 48, 16):
                            a = sfid[pl.ds(32 + off, 16)]
                            b = slv[pl.ds(32 + off, 16)]
                            sfid[pl.ds(off, 16)] = a
                            slv[pl.ds(off, 16)] = b
                    dov = ptrv >= 32
                    ptrv = jnp.where(dov, ptrv - 32, ptrv)
                    fcnt = jnp.where(do, fcnt + 1, fcnt)
                    return ptrv, fcnt
                return lax.fori_loop(0, ng, group_body, carry)

            zv = jnp.zeros((16,), jnp.int32)
            ptrv, fcnt = lax.fori_loop(0, nchunks, chunk_body,
                                       (zv, jnp.int32(0)))
            ptr = ptrv[0]

            @pl.when(ptr > 0)
            def _():
                for off in range(0, 32, 16):
                    m = (off + iota) < ptr
                    fv = jnp.where(m, sfid[pl.ds(off, 16)], 0)
                    lvv = jnp.where(m, slv[pl.ds(off, 16)], _DUMMY)
                    sfid[pl.ds(off, 16)] = fv
                    slv[pl.ds(off, 16)] = lvv
                fire(fcnt)
            fcnt = fcnt + (ptr > 0).astype(jnp.int32)

            for back in range(4, 0, -1):
                for q in range(4):
                    @pl.when((fcnt >= back) & (((fcnt - back) & 3) == q))
                    def _(q=q):
                        drain_half(q)
            plsc.subcore_barrier()

            def w(j, carry):
                i = sid + j * 16

                @pl.when(i < _VPP // 128)
                def _():
                    pltpu.sync_copy(acc.at[pl.ds(i * 128, 128)],
                                    out_hbm.at[pl.ds(gbase + i * 128, 128)])
                return carry
            lax.fori_loop(0, 7, w, 0)
            plsc.subcore_barrier()

    return k(tmp, face_t)


def _vertex_stage(acc, cnt3, depth_weights, biases):
    grid = (_NV // _BV,)
    return pl.pallas_call(
        _vert_body,
        grid=grid,
        in_specs=[
            pl.BlockSpec((_BV, _CIN), lambda i: (i, 0)),
            pl.BlockSpec((1, 1, _BV), lambda i: (i, 0, 0)),
            pl.BlockSpec((_CIN, _COUT), lambda i: (0, 0)),
            pl.BlockSpec((1, _COUT), lambda i: (0, 0)),
        ],
        out_specs=[
            pl.BlockSpec((_BV, _COUT), lambda i: (i, 0)),
            pl.BlockSpec((8, _COUT), lambda i: (0, 0)),
        ],
        out_shape=[
            jax.ShapeDtypeStruct((_NV, _COUT), jnp.float32),
            jax.ShapeDtypeStruct((8, _COUT), jnp.float32),
        ],
    )(acc, cnt3, depth_weights, biases)


def _normalize(pre, stats, gamma, beta):
    grid = (_NV // _BV,)
    return pl.pallas_call(
        _norm_body,
        grid=grid,
        in_specs=[
            pl.BlockSpec((_BV, _COUT), lambda i: (i, 0)),
            pl.BlockSpec((8, _COUT), lambda i: (0, 0)),
            pl.BlockSpec((1, _COUT), lambda i: (0, 0)),
            pl.BlockSpec((1, _COUT), lambda i: (0, 0)),
        ],
        out_specs=pl.BlockSpec((_BV, _COUT), lambda i: (i, 0)),
        out_shape=jax.ShapeDtypeStruct((_NV, _COUT), jnp.float32),
    )(pre, stats, gamma, beta)


def kernel(inputs, face, nf_count, vt_map, filt_coeff, spatial_weights,
           depth_weights, biases, gamma, beta):
    del vt_map
    sw2d = spatial_weights.reshape(_K, _CIN)
    tmp = _facet_weight(inputs, filt_coeff, sw2d)

    face_t = jnp.pad(face.T, ((0, 0), (0, _FPAD - _NF)))
    face_c = face_t.reshape(3, _NSLOT, _CCH).transpose(1, 0, 2).reshape(-1)
    acc = _sc_scatter(tmp, face_c)

    cnt3 = nf_count.reshape(_NV // _BV, 1, _BV)
    pre, stats = _vertex_stage(acc, cnt3, depth_weights, biases)
    out = _normalize(pre, stats, gamma.reshape(1, _COUT), beta.reshape(1, _COUT))
    return out
